# Initial kernel scaffold; baseline (speedup 1.0000x reference)
import jax
import jax.numpy as jnp
from jax.experimental import pallas as pl


def kernel(x, graph_edge_index, edge_index, W1, b1, W2, b2, Wc, bc):
    raise NotImplementedError("write your pallas kernel here")


# speedup vs baseline: 4.7595x; 4.7595x over previous
import functools

import jax
import jax.numpy as jnp
from jax import lax
from jax.experimental import pallas as pl
from jax.experimental.pallas import tpu as pltpu
from jax.experimental.pallas import tpu_sc as plsc

N = 10000
E = 320000
D = 128
RES = 0.1

NC, NS, L = 2, 16, 16
NW = NC * NS
EPT = E // NW
VPT = EPT // L
CB = 128
CPT = 80
NCHUNK = NW * CPT
EPAD = NCHUNK * CB
NPAD = 10240
RPT = NPAD // NS

_mesh = plsc.VectorSubcoreMesh(
    core_axis_name="c", subcore_axis_name="s", num_cores=NC, num_subcores=NS
)
_sc_params = pltpu.CompilerParams(needs_layout_passes=False)


@functools.partial(
    pl.kernel,
    out_type=jax.ShapeDtypeStruct((2 * NW, N), jnp.float32),
    mesh=_mesh,
    compiler_params=_sc_params,
    scratch_types=[
        pltpu.VMEM((N,), jnp.float32),
        pltpu.VMEM((N,), jnp.float32),
        pltpu.VMEM((EPT,), jnp.int32),
        pltpu.VMEM((EPT,), jnp.int32),
    ],
)
def _deg_kernel(src_hbm, dst_hbm, zeros1_hbm, out_hbm, accs, accd, sbuf, dbuf):
    wid = lax.axis_index("s") * NC + lax.axis_index("c")
    base = wid * EPT
    pltpu.sync_copy(zeros1_hbm, accs)
    pltpu.sync_copy(zeros1_hbm, accd)
    pltpu.sync_copy(src_hbm.at[pl.ds(base, EPT)], sbuf)
    pltpu.sync_copy(dst_hbm.at[pl.ds(base, EPT)], dbuf)
    ones = jnp.full((L,), 1.0, jnp.float32)

    def body(i, carry):
        s = sbuf[pl.ds(i * L, L)]
        d = dbuf[pl.ds(i * L, L)]
        plsc.addupdate_scatter(accs, [s], ones)
        plsc.addupdate_scatter(accd, [d], ones)
        return carry

    lax.fori_loop(0, VPT, body, 0)
    pltpu.sync_copy(accs, out_hbm.at[wid])
    pltpu.sync_copy(accd, out_hbm.at[NW + wid])


@functools.partial(
    pl.kernel,
    out_type=jax.ShapeDtypeStruct((NC, NPAD, D), jnp.float32),
    mesh=_mesh,
    compiler_params=_sc_params,
    scratch_types=[
        pltpu.VMEM_SHARED((NPAD, D), jnp.float32),
        pltpu.VMEM((CPT, CB), jnp.int32),
        pltpu.VMEM((CPT, CB), jnp.int32),
        pltpu.VMEM((CB, D), jnp.float32),
        pltpu.SemaphoreType.DMA,
    ],
)
def _agg_kernel(table_hbm, srcp_hbm, dstp_hbm, zeros2_hbm, out_hbm,
                acc, sidx, didx, rows, sem):
    cid = lax.axis_index("c")
    sid = lax.axis_index("s")
    wid = sid * NC + cid
    pltpu.sync_copy(zeros2_hbm, acc.at[pl.ds(sid * RPT, RPT)])
    pltpu.sync_copy(srcp_hbm.at[pl.ds(wid * CPT, CPT)], sidx)
    pltpu.sync_copy(dstp_hbm.at[pl.ds(wid * CPT, CPT)], didx)
    plsc.subcore_barrier()

    def body(j, carry):
        pltpu.async_copy(table_hbm.at[sidx.at[j]], rows, sem).wait()
        pltpu.sync_copy(rows, acc.at[didx.at[j]], add=True)
        return carry

    lax.fori_loop(0, CPT, body, 0)
    plsc.subcore_barrier()
    pltpu.sync_copy(acc.at[pl.ds(sid * RPT, RPT)],
                    out_hbm.at[cid, pl.ds(sid * RPT, RPT)])


@functools.partial(
    pl.kernel,
    out_type=jax.ShapeDtypeStruct((E,), jnp.float32),
    mesh=_mesh,
    compiler_params=_sc_params,
    scratch_types=[
        pltpu.VMEM((N,), jnp.float32),
        pltpu.VMEM((N,), jnp.float32),
        pltpu.VMEM((EPT,), jnp.int32),
        pltpu.VMEM((EPT,), jnp.int32),
        pltpu.VMEM((EPT,), jnp.float32),
    ],
)
def _logit_kernel(u_hbm, v_hbm, es_hbm, ed_hbm, out_hbm,
                  ub, vb, sbuf, dbuf, obuf):
    wid = lax.axis_index("s") * NC + lax.axis_index("c")
    base = wid * EPT
    pltpu.sync_copy(u_hbm, ub)
    pltpu.sync_copy(v_hbm, vb)
    pltpu.sync_copy(es_hbm.at[pl.ds(base, EPT)], sbuf)
    pltpu.sync_copy(ed_hbm.at[pl.ds(base, EPT)], dbuf)
    one = jnp.full((L,), 1.0, jnp.float32)

    def body(i, carry):
        s = sbuf[pl.ds(i * L, L)]
        d = dbuf[pl.ds(i * L, L)]
        lu = plsc.load_gather(ub, [s])
        lv = plsc.load_gather(vb, [d])
        obuf[pl.ds(i * L, L)] = one / (one + jnp.exp(-(lu + lv)))
        return carry

    lax.fori_loop(0, VPT, body, 0)
    pltpu.sync_copy(obuf, out_hbm.at[pl.ds(base, EPT)])


def _norm_from(degp, lo):
    deg = jnp.sum(degp[lo:lo + NW], axis=0)
    return jnp.where(deg > 0, 1.0 / jnp.sqrt(jnp.maximum(deg, 1.0)), 0.0)


def _k2_body(degp_ref, x_ref, w1_ref, out_ref):
    ns = _norm_from(degp_ref[...], 0)
    t = jnp.dot(x_ref[...], w1_ref[...], preferred_element_type=jnp.float32)
    out_ref[pl.ds(0, N), :] = t * ns[:, None]
    out_ref[pl.ds(N, NPAD - N), :] = jnp.zeros((NPAD - N, D), jnp.float32)


def _k4_body(parts_ref, degp_ref, x_ref, w2_ref, b1_ref, out_ref):
    p = parts_ref[...]
    degp = degp_ref[...]
    ns = _norm_from(degp, 0)
    nd = _norm_from(degp, NW)
    h1 = (p[0, :N] + p[1, :N]) * nd[:, None] + b1_ref[...] + RES * x_ref[...]
    h1 = jnp.maximum(h1, 0.0)
    t = jnp.dot(h1, w2_ref[...], preferred_element_type=jnp.float32)
    out_ref[pl.ds(0, N), :] = t * ns[:, None]
    out_ref[pl.ds(N, NPAD - N), :] = jnp.zeros((NPAD - N, D), jnp.float32)


def _k6_body(parts_ref, degp_ref, x_ref, b2_ref, wc1_ref, wc2_ref, bc_ref,
             z_ref, u_ref, v_ref):
    p = parts_ref[...]
    nd = _norm_from(degp_ref[...], NW)
    z = (p[0, :N] + p[1, :N]) * nd[:, None] + b2_ref[...] + RES * x_ref[...]
    z_ref[...] = z
    u_ref[...] = jnp.dot(z, wc1_ref[...],
                         preferred_element_type=jnp.float32) + bc_ref[0, 0]
    v_ref[...] = jnp.dot(z, wc2_ref[...], preferred_element_type=jnp.float32)


_k2 = pl.pallas_call(
    _k2_body, out_shape=jax.ShapeDtypeStruct((NPAD, D), jnp.float32))
_k4 = pl.pallas_call(
    _k4_body, out_shape=jax.ShapeDtypeStruct((NPAD, D), jnp.float32))
_k6 = pl.pallas_call(
    _k6_body,
    out_shape=(
        jax.ShapeDtypeStruct((N, D), jnp.float32),
        jax.ShapeDtypeStruct((N, 1), jnp.float32),
        jax.ShapeDtypeStruct((N, 1), jnp.float32),
    ),
)


def kernel(x, graph_edge_index, edge_index, W1, b1, W2, b2, Wc, bc):
    src = graph_edge_index[0]
    dst = graph_edge_index[1]
    es = edge_index[0]
    ed = edge_index[1]
    zeros1 = jnp.zeros((N,), jnp.float32)
    zeros2 = jnp.zeros((RPT, D), jnp.float32)
    pad = jnp.full((EPAD - E,), N, jnp.int32)
    srcp = jnp.concatenate([src, pad]).reshape(NCHUNK, CB)
    dstp = jnp.concatenate([dst, pad]).reshape(NCHUNK, CB)

    degp = _deg_kernel(src, dst, zeros1)
    t1 = _k2(degp, x, W1)
    parts1 = _agg_kernel(t1, srcp, dstp, zeros2)
    t2 = _k4(parts1, degp, x, W2, b1.reshape(1, D))
    parts2 = _agg_kernel(t2, srcp, dstp, zeros2)
    z, u, v = _k6(parts2, degp, x, b2.reshape(1, D),
                  Wc[:D], Wc[D:], bc.reshape(1, 1))
    sig = _logit_kernel(u.reshape(N), v.reshape(N), es, ed)
    return z, sig.reshape(E, 1)
